# Initial kernel scaffold; baseline (speedup 1.0000x reference)
import functools

import jax
import jax.numpy as jnp
from jax import lax
from jax.experimental import pallas as pl
from jax.experimental.pallas import tpu as pltpu
from jax.experimental.pallas import tpu_sc as plsc

N = 10000
D = 128
E = 320000

NC = 2
NS = 16
NW = NC * NS
K = 128
EPW = -(-(E // NW) // K) * K
NB = EPW // K
E_PAD = NW * EPW
N_PAD = 10240
RPS = N_PAD // NS
TCB = 256


def _mlp_relu_body(x_ref, w_ref, b_ref, o_ref):
    o_ref[:] = jax.nn.relu(
        jnp.dot(x_ref[:], w_ref[:], preferred_element_type=jnp.float32)
        + b_ref[:]
    )


def _msg_precompute(x_pad, Wp, bp2):
    return pl.pallas_call(
        _mlp_relu_body,
        grid=(N_PAD // TCB,),
        in_specs=[
            pl.BlockSpec((TCB, D), lambda i: (i, 0)),
            pl.BlockSpec((D, D), lambda i: (0, 0)),
            pl.BlockSpec((1, D), lambda i: (0, 0)),
        ],
        out_specs=pl.BlockSpec((TCB, D), lambda i: (i, 0)),
        out_shape=jax.ShapeDtypeStruct((N_PAD, D), jnp.float32),
    )(x_pad, Wp, bp2)


def _sc_edge_body(m_hbm, src_hbm, dst_hbm, z_out, deg_out,
                  src_v, dst_v, rows_v, ones_v, zsh, degsh, sem):
    c = lax.axis_index("c")
    s = lax.axis_index("s")
    w = c * NS + s

    zero16 = jnp.zeros((16,), jnp.float32)
    one16 = jnp.ones((16,), jnp.float32)

    def _zrow(i, _):
        def _zcol(j, _):
            rows_v[i, pl.ds(j * 16, 16)] = zero16
            return 0
        lax.fori_loop(0, D // 16, _zcol, 0)
        ones_v[i, :] = zero16
        return 0
    lax.fori_loop(0, K, _zrow, 0)

    def _zacc(k, _):
        pltpu.sync_copy(rows_v, zsh.at[pl.ds(s * RPS + k * K, K)])
        pltpu.sync_copy(ones_v, degsh.at[pl.ds(s * RPS + k * K, K)])
        return 0
    lax.fori_loop(0, RPS // K, _zacc, 0)

    def _ones(i, _):
        ones_v[i, :] = one16
        return 0
    lax.fori_loop(0, K, _ones, 0)

    pltpu.sync_copy(src_hbm.at[w], src_v)
    pltpu.sync_copy(dst_hbm.at[w], dst_v)

    plsc.subcore_barrier()

    def _blk(j, _):
        pltpu.async_copy(m_hbm.at[src_v.at[j]], rows_v, sem).wait()
        pltpu.sync_copy(rows_v, zsh.at[dst_v.at[j]], add=True)
        pltpu.sync_copy(ones_v, degsh.at[dst_v.at[j]], add=True)
        return 0
    lax.fori_loop(0, NB, _blk, 0)

    plsc.subcore_barrier()

    sl = pl.ds(s * RPS, RPS)
    pltpu.sync_copy(zsh.at[sl], z_out.at[c, sl])
    pltpu.sync_copy(degsh.at[sl], deg_out.at[c, sl])


def _sc_edge_pass(m_pad, src3, dst3):
    mesh = plsc.VectorSubcoreMesh(
        core_axis_name="c", subcore_axis_name="s",
        num_cores=NC, num_subcores=NS,
    )
    return pl.kernel(
        _sc_edge_body,
        out_type=(
            jax.ShapeDtypeStruct((NC, N_PAD, D), jnp.float32),
            jax.ShapeDtypeStruct((NC, N_PAD, 16), jnp.float32),
        ),
        mesh=mesh,
        scratch_types=[
            pltpu.VMEM((NB, K), jnp.int32),
            pltpu.VMEM((NB, K), jnp.int32),
            pltpu.VMEM((K, D), jnp.float32),
            pltpu.VMEM((K, 16), jnp.float32),
            pltpu.VMEM_SHARED((N_PAD, D), jnp.float32),
            pltpu.VMEM_SHARED((N_PAD, 16), jnp.float32),
            pltpu.SemaphoreType.DMA,
        ],
    )(m_pad, src3, dst3)


def _update_body(za_ref, zb_ref, da_ref, db_ref, x_ref, w_ref, b_ref, o_ref):
    deg = jnp.maximum(da_ref[0, :, 0:1] + db_ref[0, :, 0:1], 1.0)
    z = (za_ref[0] + zb_ref[0]) / deg
    o_ref[:] = jax.nn.relu(
        jnp.dot(z, w_ref[:], preferred_element_type=jnp.float32) + b_ref[:]
    ) + x_ref[:]


def _node_update(z2, deg2, x_pad, Wu, bu2):
    return pl.pallas_call(
        _update_body,
        grid=(N_PAD // TCB,),
        in_specs=[
            pl.BlockSpec((1, TCB, D), lambda i: (0, i, 0)),
            pl.BlockSpec((1, TCB, D), lambda i: (1, i, 0)),
            pl.BlockSpec((1, TCB, 16), lambda i: (0, i, 0)),
            pl.BlockSpec((1, TCB, 16), lambda i: (1, i, 0)),
            pl.BlockSpec((TCB, D), lambda i: (i, 0)),
            pl.BlockSpec((D, D), lambda i: (0, 0)),
            pl.BlockSpec((1, D), lambda i: (0, 0)),
        ],
        out_specs=pl.BlockSpec((TCB, D), lambda i: (i, 0)),
        out_shape=jax.ShapeDtypeStruct((N_PAD, D), jnp.float32),
    )(z2, z2, deg2, deg2, x_pad, Wu, bu2)


def kernel(x, edge_index, Wp, bp, Wu, bu):
    x = x.astype(jnp.float32)
    x_pad = jnp.pad(x, ((0, N_PAD - N), (0, 0)))

    src = edge_index[0].astype(jnp.int32)
    dst = edge_index[1].astype(jnp.int32)
    pad = E_PAD - E
    fill = jnp.full((pad,), N, jnp.int32)
    src3 = jnp.concatenate([src, fill]).reshape(NW, NB, K)
    dst3 = jnp.concatenate([dst, fill]).reshape(NW, NB, K)

    m_pad = _msg_precompute(x_pad, Wp, bp.reshape(1, D))
    z2, deg2 = _sc_edge_pass(m_pad, src3, dst3)
    h_pad = _node_update(z2, deg2, x_pad, Wu, bu.reshape(1, D))
    return h_pad[:N]


# speedup vs baseline: 5.3712x; 5.3712x over previous
import jax
import jax.numpy as jnp
from jax import lax
from jax.experimental import pallas as pl
from jax.experimental.pallas import tpu as pltpu
from jax.experimental.pallas import tpu_sc as plsc

N = 10000
D = 128
E = 320000

NC = 2
NS = 16
K = 128
EPW = -(-(E // NS) // K) * K
NB = EPW // K
E_PAD = NS * EPW
N_PAD = 10240
NH = N_PAD // NC
ZROWS = NH + K
RPC = NH // NS
TCB = 256


def _mlp_relu_body(x_ref, w_ref, b_ref, o_ref):
    o_ref[:] = jax.nn.relu(
        jnp.dot(x_ref[:], w_ref[:], preferred_element_type=jnp.float32)
        + b_ref[:]
    )


def _msg_precompute(x_pad, Wp, bp2):
    return pl.pallas_call(
        _mlp_relu_body,
        grid=(N_PAD // TCB,),
        in_specs=[
            pl.BlockSpec((TCB, D), lambda i: (i, 0)),
            pl.BlockSpec((D, D), lambda i: (0, 0)),
            pl.BlockSpec((1, D), lambda i: (0, 0)),
        ],
        out_specs=pl.BlockSpec((TCB, D), lambda i: (i, 0)),
        out_shape=jax.ShapeDtypeStruct((N_PAD, D), jnp.float32),
    )(x_pad, Wp, bp2)


def _sc_edge_body(m_hbm, src_hbm, dst_hbm, z_out, deg_out,
                  src_v, dst_v, dstm_v, rows_v, hist_v, zsh, sem):
    c = lax.axis_index("c")
    s = lax.axis_index("s")

    zero16 = jnp.zeros((16,), jnp.float32)
    one16 = jnp.ones((16,), jnp.float32)

    def _zrow(i, _):
        def _zcol(j, _):
            rows_v[i, pl.ds(j * 16, 16)] = zero16
            return 0
        lax.fori_loop(0, D // 16, _zcol, 0)
        return 0
    lax.fori_loop(0, K, _zrow, 0)

    def _zhist(i, _):
        hist_v[pl.ds(i * 16, 16)] = zero16
        return 0
    lax.fori_loop(0, N_PAD // 16, _zhist, 0)

    nzch = ZROWS // K
    def _zacc(k, _):
        ch = k * NS + s

        @pl.when(ch < nzch)
        def _():
            pltpu.sync_copy(rows_v, zsh.at[pl.ds(ch * K, K)])
        return 0
    lax.fori_loop(0, -(-nzch // NS), _zacc, 0)

    pltpu.sync_copy(src_hbm.at[s], src_v)
    pltpu.sync_copy(dst_hbm.at[s], dst_v)

    lane = lax.iota(jnp.int32, 16)
    base = c * NH

    def _remap(i, _):
        def _rcol(j, _):
            v = dst_v[i, pl.ds(j * 16, 16)]
            local = v - base
            ok = (local >= 0) & (local < NH)
            dummy = NH + lane + (((i * 8 + j) % 8) * 16)
            dstm_v[i, pl.ds(j * 16, 16)] = jnp.where(ok, local, dummy)
            return 0
        lax.fori_loop(0, K // 16, _rcol, 0)
        return 0
    lax.fori_loop(0, NB, _remap, 0)

    plsc.subcore_barrier()

    def _blk(j, _):
        cp = pltpu.async_copy(m_hbm.at[src_v.at[j]], rows_v, sem)

        @pl.when(c == 0)
        def _():
            def _hist(t, _):
                idx = dst_v[j, pl.ds(t * 16, 16)]
                plsc.addupdate_scatter(hist_v, [idx], one16)
                return 0
            lax.fori_loop(0, K // 16, _hist, 0)

        cp.wait()
        pltpu.sync_copy(rows_v, zsh.at[dstm_v.at[j]], add=True)
        return 0
    lax.fori_loop(0, NB, _blk, 0)

    plsc.subcore_barrier()

    sl = pl.ds(s * RPC, RPC)
    pltpu.sync_copy(zsh.at[sl], z_out.at[pl.ds(base + s * RPC, RPC)])

    @pl.when(c == 0)
    def _():
        pltpu.sync_copy(hist_v, deg_out.at[s])


def _sc_edge_pass(m_pad, src3, dst3):
    mesh = plsc.VectorSubcoreMesh(
        core_axis_name="c", subcore_axis_name="s",
        num_cores=NC, num_subcores=NS,
    )
    return pl.kernel(
        _sc_edge_body,
        out_type=(
            jax.ShapeDtypeStruct((N_PAD, D), jnp.float32),
            jax.ShapeDtypeStruct((NS, N_PAD), jnp.float32),
        ),
        mesh=mesh,
        scratch_types=[
            pltpu.VMEM((NB, K), jnp.int32),
            pltpu.VMEM((NB, K), jnp.int32),
            pltpu.VMEM((NB, K), jnp.int32),
            pltpu.VMEM((K, D), jnp.float32),
            pltpu.VMEM((N_PAD,), jnp.float32),
            pltpu.VMEM_SHARED((ZROWS, D), jnp.float32),
            pltpu.SemaphoreType.DMA,
        ],
        compiler_params=pltpu.CompilerParams(needs_layout_passes=False),
    )(m_pad, src3, dst3)


def _update_body(z_ref, d_ref, x_ref, w_ref, b_ref, o_ref):
    ones_cols = jnp.ones((NS, D), jnp.float32)
    deg_mat = lax.dot_general(
        d_ref[:], ones_cols, (((0,), (0,)), ((), ())),
        preferred_element_type=jnp.float32,
    )
    z = z_ref[:] / jnp.maximum(deg_mat, 1.0)
    o_ref[:] = jax.nn.relu(
        jnp.dot(z, w_ref[:], preferred_element_type=jnp.float32) + b_ref[:]
    ) + x_ref[:]


def _node_update(z, deg, x_pad, Wu, bu2):
    return pl.pallas_call(
        _update_body,
        grid=(N_PAD // TCB,),
        in_specs=[
            pl.BlockSpec((TCB, D), lambda i: (i, 0)),
            pl.BlockSpec((NS, TCB), lambda i: (0, i)),
            pl.BlockSpec((TCB, D), lambda i: (i, 0)),
            pl.BlockSpec((D, D), lambda i: (0, 0)),
            pl.BlockSpec((1, D), lambda i: (0, 0)),
        ],
        out_specs=pl.BlockSpec((TCB, D), lambda i: (i, 0)),
        out_shape=jax.ShapeDtypeStruct((N_PAD, D), jnp.float32),
    )(z, deg, x_pad, Wu, bu2)


def kernel(x, edge_index, Wp, bp, Wu, bu):
    x = x.astype(jnp.float32)
    x_pad = jnp.pad(x, ((0, N_PAD - N), (0, 0)))

    src = edge_index[0].astype(jnp.int32)
    dst = edge_index[1].astype(jnp.int32)
    pad = E_PAD - E
    fill = N + (jnp.arange(pad, dtype=jnp.int32) % K)
    src3 = jnp.concatenate([src, fill]).reshape(NS, NB, K)
    dst3 = jnp.concatenate([dst, fill]).reshape(NS, NB, K)

    m_pad = _msg_precompute(x_pad, Wp, bp.reshape(1, D))
    z, deg = _sc_edge_pass(m_pad, src3, dst3)
    h_pad = _node_update(z, deg, x_pad, Wu, bu.reshape(1, D))
    return h_pad[:N]


# speedup vs baseline: 6.1654x; 1.1478x over previous
import jax
import jax.numpy as jnp
from jax import lax
from jax.experimental import pallas as pl
from jax.experimental.pallas import tpu as pltpu
from jax.experimental.pallas import tpu_sc as plsc

N = 10000
D = 128
E = 320000

NC = 2
NS = 16
K = 128
NB = -(-(E // NS) // K)
EPW = NB * K
E_PAD = NS * EPW
N_PAD = 10240
NH = N_PAD // NC
ZROWS = NH + K
RPC = NH // NS
TCB = 256


def _mlp_relu_body(x_ref, w_ref, b_ref, o_ref):
    o_ref[:] = jax.nn.relu(
        jnp.dot(x_ref[:], w_ref[:], preferred_element_type=jnp.float32)
        + b_ref[:]
    )


def _msg_precompute(x_pad, Wp, bp2):
    return pl.pallas_call(
        _mlp_relu_body,
        grid=(N_PAD // TCB,),
        in_specs=[
            pl.BlockSpec((TCB, D), lambda i: (i, 0)),
            pl.BlockSpec((D, D), lambda i: (0, 0)),
            pl.BlockSpec((1, D), lambda i: (0, 0)),
        ],
        out_specs=pl.BlockSpec((TCB, D), lambda i: (i, 0)),
        out_shape=jax.ShapeDtypeStruct((N_PAD, D), jnp.float32),
    )(x_pad, Wp, bp2)


def _sc_edge_body(m_hbm, src_hbm, dst_hbm, z_out, deg_out,
                  src_v, dst_v, rows2, hist_v, zsh, sems):
    c = lax.axis_index("c")
    s = lax.axis_index("s")

    zero16 = jnp.zeros((16,), jnp.float32)
    one16 = jnp.ones((16,), jnp.float32)

    def _zrow(i, _):
        def _zcol(j, _):
            rows2[0, i, pl.ds(j * 16, 16)] = zero16
            return 0
        lax.fori_loop(0, D // 16, _zcol, 0)
        return 0
    lax.fori_loop(0, K, _zrow, 0)

    def _zhist(i, _):
        hist_v[pl.ds(i * 16, 16)] = zero16
        return 0
    lax.fori_loop(0, NH // 16, _zhist, 0)

    nzch = ZROWS // K
    def _zacc(k, _):
        ch = k * NS + s

        @pl.when(ch < nzch)
        def _():
            pltpu.sync_copy(rows2.at[0], zsh.at[pl.ds(ch * K, K)])
        return 0
    lax.fori_loop(0, -(-nzch // NS), _zacc, 0)

    pltpu.sync_copy(src_hbm.at[s], src_v)
    pltpu.sync_copy(dst_hbm.at[s], dst_v)

    lane = lax.iota(jnp.int32, 16)
    base = c * NH

    def _remap(i, _):
        def _rcol(j, _):
            v = dst_v[i, pl.ds(j * 16, 16)]
            local = v - base
            ok = (local >= 0) & (local < NH)
            dummy = NH + lane + (((i * 8 + j) % 8) * 16)
            dst_v[i, pl.ds(j * 16, 16)] = jnp.where(ok, local, dummy)
            return 0
        lax.fori_loop(0, K // 16, _rcol, 0)
        return 0
    lax.fori_loop(0, NB, _remap, 0)

    plsc.subcore_barrier()

    nh2 = NB // 2

    def _blk(i, _):
        ja = 2 * i
        jb = 2 * i + 1
        da = pltpu.async_copy(m_hbm.at[src_v.at[ja]], rows2.at[0],
                              sems.at[0])
        db = pltpu.async_copy(m_hbm.at[src_v.at[jb]], rows2.at[1],
                              sems.at[1])

        def _hist(t, _):
            idx = dst_v[ja + t // 8, pl.ds((t % 8) * 16, 16)]
            plsc.addupdate_scatter(hist_v, [idx], one16, mask=idx < NH)
            return 0
        lax.fori_loop(0, 2 * (K // 16), _hist, 0)

        da.wait()
        pltpu.sync_copy(rows2.at[0], zsh.at[dst_v.at[ja]], add=True)
        db.wait()
        pltpu.sync_copy(rows2.at[1], zsh.at[dst_v.at[jb]], add=True)
        return 0
    lax.fori_loop(0, nh2, _blk, 0)

    @pl.when(NB % 2 == 1)
    def _():
        j = NB - 1
        cp = pltpu.async_copy(m_hbm.at[src_v.at[j]], rows2.at[0],
                              sems.at[0])

        def _hist(t, _):
            idx = dst_v[j, pl.ds(t * 16, 16)]
            plsc.addupdate_scatter(hist_v, [idx], one16, mask=idx < NH)
            return 0
        lax.fori_loop(0, K // 16, _hist, 0)

        cp.wait()
        pltpu.sync_copy(rows2.at[0], zsh.at[dst_v.at[j]], add=True)

    plsc.subcore_barrier()

    sl = pl.ds(s * RPC, RPC)
    pltpu.sync_copy(zsh.at[sl], z_out.at[pl.ds(base + s * RPC, RPC)])
    pltpu.sync_copy(hist_v, deg_out.at[c, s])


def _sc_edge_pass(m_pad, src3, dst3):
    mesh = plsc.VectorSubcoreMesh(
        core_axis_name="c", subcore_axis_name="s",
        num_cores=NC, num_subcores=NS,
    )
    return pl.kernel(
        _sc_edge_body,
        out_type=(
            jax.ShapeDtypeStruct((N_PAD, D), jnp.float32),
            jax.ShapeDtypeStruct((NC, NS, NH), jnp.float32),
        ),
        mesh=mesh,
        scratch_types=[
            pltpu.VMEM((NB, K), jnp.int32),
            pltpu.VMEM((NB, K), jnp.int32),
            pltpu.VMEM((2, K, D), jnp.float32),
            pltpu.VMEM((NH,), jnp.float32),
            pltpu.VMEM_SHARED((ZROWS, D), jnp.float32),
            pltpu.SemaphoreType.DMA((2,)),
        ],
        compiler_params=pltpu.CompilerParams(needs_layout_passes=False),
    )(m_pad, src3, dst3)


def _update_body(z_ref, d_ref, x_ref, w_ref, b_ref, o_ref):
    ones_cols = jnp.ones((NS, D), jnp.float32)
    deg_mat = lax.dot_general(
        d_ref[0], ones_cols, (((0,), (0,)), ((), ())),
        preferred_element_type=jnp.float32,
    )
    z = z_ref[:] / jnp.maximum(deg_mat, 1.0)
    o_ref[:] = jax.nn.relu(
        jnp.dot(z, w_ref[:], preferred_element_type=jnp.float32) + b_ref[:]
    ) + x_ref[:]


def _node_update(z, deg, x_pad, Wu, bu2):
    return pl.pallas_call(
        _update_body,
        grid=(N_PAD // TCB,),
        in_specs=[
            pl.BlockSpec((TCB, D), lambda i: (i, 0)),
            pl.BlockSpec(
                (1, NS, TCB),
                lambda i: (i // (NH // TCB), 0, i % (NH // TCB)),
            ),
            pl.BlockSpec((TCB, D), lambda i: (i, 0)),
            pl.BlockSpec((D, D), lambda i: (0, 0)),
            pl.BlockSpec((1, D), lambda i: (0, 0)),
        ],
        out_specs=pl.BlockSpec((TCB, D), lambda i: (i, 0)),
        out_shape=jax.ShapeDtypeStruct((N_PAD, D), jnp.float32),
    )(z, deg, x_pad, Wu, bu2)


def kernel(x, edge_index, Wp, bp, Wu, bu):
    x = x.astype(jnp.float32)
    x_pad = jnp.pad(x, ((0, N_PAD - N), (0, 0)))

    src = edge_index[0].astype(jnp.int32)
    dst = edge_index[1].astype(jnp.int32)
    pad = E_PAD - E
    fill = N + (jnp.arange(pad, dtype=jnp.int32) % K)
    src3 = jnp.concatenate([src, fill]).reshape(NS, NB, K)
    dst3 = jnp.concatenate([dst, fill]).reshape(NS, NB, K)

    m_pad = _msg_precompute(x_pad, Wp, bp.reshape(1, D))
    z, deg = _sc_edge_pass(m_pad, src3, dst3)
    h_pad = _node_update(z, deg, x_pad, Wu, bu.reshape(1, D))
    return h_pad[:N]


# speedup vs baseline: 6.3076x; 1.0231x over previous
import jax
import jax.numpy as jnp
from jax import lax
from jax.experimental import pallas as pl
from jax.experimental.pallas import tpu as pltpu
from jax.experimental.pallas import tpu_sc as plsc

N = 10000
D = 128
E = 320000

NC = 2
NS = 16
K = 128
NB = -(-(E // NS) // K)
EPW = NB * K
E_PAD = NS * EPW
N_PAD = 10240
NH = N_PAD // NC
ZROWS = NH + K
RPC = NH // NS
TCB = 256


def _mlp_relu_body(x_ref, w_ref, b_ref, o_ref):
    o_ref[:] = jax.nn.relu(
        jnp.dot(x_ref[:], w_ref[:], preferred_element_type=jnp.float32)
        + b_ref[:]
    )


def _msg_precompute(x, Wp, bp2):
    return pl.pallas_call(
        _mlp_relu_body,
        grid=(N_PAD // TCB,),
        in_specs=[
            pl.BlockSpec((TCB, D), lambda i: (i, 0)),
            pl.BlockSpec((D, D), lambda i: (0, 0)),
            pl.BlockSpec((1, D), lambda i: (0, 0)),
        ],
        out_specs=pl.BlockSpec((TCB, D), lambda i: (i, 0)),
        out_shape=jax.ShapeDtypeStruct((N_PAD, D), jnp.float32),
    )(x, Wp, bp2)


def _sc_edge_body(m_hbm, src_hbm, dst_hbm, z_out, deg_out,
                  src_v, dst_v, rows2, hist_v, zsh, sems):
    c = lax.axis_index("c")
    s = lax.axis_index("s")

    zero16 = jnp.zeros((16,), jnp.float32)
    one16 = jnp.ones((16,), jnp.float32)

    def _zrow(i, _):
        def _zcol(j, _):
            rows2[0, i, pl.ds(j * 16, 16)] = zero16
            return 0
        lax.fori_loop(0, D // 16, _zcol, 0)
        return 0
    lax.fori_loop(0, K, _zrow, 0)

    def _zhist(i, _):
        hist_v[pl.ds(i * 16, 16)] = zero16
        return 0
    lax.fori_loop(0, NH // 16, _zhist, 0)

    nzch = ZROWS // K
    def _zacc(k, _):
        ch = k * NS + s

        @pl.when(ch < nzch)
        def _():
            pltpu.sync_copy(rows2.at[0], zsh.at[pl.ds(ch * K, K)])
        return 0
    lax.fori_loop(0, -(-nzch // NS), _zacc, 0)

    pltpu.sync_copy(src_hbm.at[s], src_v)
    pltpu.sync_copy(dst_hbm.at[s], dst_v)

    lane = lax.iota(jnp.int32, 16)
    base = c * NH

    def _remap(i, _):
        def _rcol(j, _):
            v = dst_v[i, pl.ds(j * 16, 16)]
            local = v - base
            ok = (local >= 0) & (local < NH)
            dummy = NH + lane + (((i * 8 + j) % 8) * 16)
            dst_v[i, pl.ds(j * 16, 16)] = jnp.where(ok, local, dummy)
            return 0
        lax.fori_loop(0, K // 16, _rcol, 0)
        return 0
    lax.fori_loop(0, NB, _remap, 0)

    plsc.subcore_barrier()

    nh2 = NB // 2

    def _blk(i, _):
        ja = 2 * i
        jb = 2 * i + 1
        da = pltpu.async_copy(m_hbm.at[src_v.at[ja]], rows2.at[0],
                              sems.at[0])
        db = pltpu.async_copy(m_hbm.at[src_v.at[jb]], rows2.at[1],
                              sems.at[1])

        def _hist(t, _):
            idx = dst_v[ja + t // 8, pl.ds((t % 8) * 16, 16)]
            plsc.addupdate_scatter(hist_v, [idx], one16, mask=idx < NH)
            return 0
        lax.fori_loop(0, 2 * (K // 16), _hist, 0)

        da.wait()
        pltpu.sync_copy(rows2.at[0], zsh.at[dst_v.at[ja]], add=True)
        db.wait()
        pltpu.sync_copy(rows2.at[1], zsh.at[dst_v.at[jb]], add=True)
        return 0
    lax.fori_loop(0, nh2, _blk, 0)

    @pl.when(NB % 2 == 1)
    def _():
        j = NB - 1
        cp = pltpu.async_copy(m_hbm.at[src_v.at[j]], rows2.at[0],
                              sems.at[0])

        def _hist(t, _):
            idx = dst_v[j, pl.ds(t * 16, 16)]
            plsc.addupdate_scatter(hist_v, [idx], one16, mask=idx < NH)
            return 0
        lax.fori_loop(0, K // 16, _hist, 0)

        cp.wait()
        pltpu.sync_copy(rows2.at[0], zsh.at[dst_v.at[j]], add=True)

    plsc.subcore_barrier()

    sl = pl.ds(s * RPC, RPC)
    pltpu.sync_copy(zsh.at[sl], z_out.at[pl.ds(base + s * RPC, RPC)])
    pltpu.sync_copy(hist_v, deg_out.at[c, s])


def _sc_edge_pass(m_pad, src3, dst3):
    mesh = plsc.VectorSubcoreMesh(
        core_axis_name="c", subcore_axis_name="s",
        num_cores=NC, num_subcores=NS,
    )
    return pl.kernel(
        _sc_edge_body,
        out_type=(
            jax.ShapeDtypeStruct((N_PAD, D), jnp.float32),
            jax.ShapeDtypeStruct((NC, NS, NH), jnp.float32),
        ),
        mesh=mesh,
        scratch_types=[
            pltpu.VMEM((NB, K), jnp.int32),
            pltpu.VMEM((NB, K), jnp.int32),
            pltpu.VMEM((2, K, D), jnp.float32),
            pltpu.VMEM((NH,), jnp.float32),
            pltpu.VMEM_SHARED((ZROWS, D), jnp.float32),
            pltpu.SemaphoreType.DMA((2,)),
        ],
        compiler_params=pltpu.CompilerParams(needs_layout_passes=False),
    )(m_pad, src3, dst3)


def _update_body(z_ref, d_ref, x_ref, w_ref, b_ref, o_ref):
    ones_cols = jnp.ones((NS, D), jnp.float32)
    deg_mat = lax.dot_general(
        d_ref[0], ones_cols, (((0,), (0,)), ((), ())),
        preferred_element_type=jnp.float32,
    )
    z = z_ref[:] / jnp.maximum(deg_mat, 1.0)
    o_ref[:] = jax.nn.relu(
        jnp.dot(z, w_ref[:], preferred_element_type=jnp.float32) + b_ref[:]
    ) + x_ref[:]


def _node_update(z, deg, x, Wu, bu2):
    return pl.pallas_call(
        _update_body,
        grid=(N_PAD // TCB,),
        in_specs=[
            pl.BlockSpec((TCB, D), lambda i: (i, 0)),
            pl.BlockSpec(
                (1, NS, TCB),
                lambda i: (i // (NH // TCB), 0, i % (NH // TCB)),
            ),
            pl.BlockSpec((TCB, D), lambda i: (i, 0)),
            pl.BlockSpec((D, D), lambda i: (0, 0)),
            pl.BlockSpec((1, D), lambda i: (0, 0)),
        ],
        out_specs=pl.BlockSpec((TCB, D), lambda i: (i, 0)),
        out_shape=jax.ShapeDtypeStruct((N, D), jnp.float32),
    )(z, deg, x, Wu, bu2)


def kernel(x, edge_index, Wp, bp, Wu, bu):
    x = x.astype(jnp.float32)

    src = edge_index[0].astype(jnp.int32)
    dst = edge_index[1].astype(jnp.int32)
    pad = E_PAD - E
    fill = N + (jnp.arange(pad, dtype=jnp.int32) % K)
    src3 = jnp.concatenate([src, fill]).reshape(NS, NB, K)
    dst3 = jnp.concatenate([dst, fill]).reshape(NS, NB, K)

    m_pad = _msg_precompute(x, Wp, bp.reshape(1, D))
    z, deg = _sc_edge_pass(m_pad, src3, dst3)
    return _node_update(z, deg, x, Wu, bu.reshape(1, D))


# speedup vs baseline: 8.3533x; 1.3243x over previous
import jax
import jax.numpy as jnp
from jax import lax
from jax.experimental import pallas as pl
from jax.experimental.pallas import tpu as pltpu
from jax.experimental.pallas import tpu_sc as plsc

N = 10000
D = 128
E = 320000

NC = 2
NS = 16
K = 128
NB = -(-(E // NS) // K)
EPW = NB * K
E_PAD = NS * EPW
N_PAD = 10240
NH = N_PAD // NC
ZROWS = NH + K
RPC = NH // NS
TCB = 256


def _mlp_relu_body(x_ref, w_ref, b_ref, o_ref):
    o_ref[:] = jax.nn.relu(
        jnp.dot(x_ref[:], w_ref[:], preferred_element_type=jnp.float32)
        + b_ref[:]
    )


def _msg_precompute(x, Wp, bp2):
    return pl.pallas_call(
        _mlp_relu_body,
        grid=(N_PAD // TCB,),
        in_specs=[
            pl.BlockSpec((TCB, D), lambda i: (i, 0)),
            pl.BlockSpec((D, D), lambda i: (0, 0)),
            pl.BlockSpec((1, D), lambda i: (0, 0)),
        ],
        out_specs=pl.BlockSpec((TCB, D), lambda i: (i, 0)),
        out_shape=jax.ShapeDtypeStruct((N_PAD, D), jnp.float32),
    )(x, Wp, bp2)


def _sc_edge_body(m_hbm, src_hbm, dst_hbm, z_out, deg_out,
                  src_v, dst_v, dst2_v, rows2, hist_v, zsh, sems):
    c = lax.axis_index("c")
    s = lax.axis_index("s")

    zero16 = jnp.zeros((16,), jnp.float32)
    one16 = jnp.ones((16,), jnp.float32)
    lane = lax.iota(jnp.int32, 16)
    base = c * NH

    def _zrow(i, _):
        def _zcol(j, _):
            rows2[0, i, pl.ds(j * 16, 16)] = zero16
            return 0
        lax.fori_loop(0, D // 16, _zcol, 0)
        return 0
    lax.fori_loop(0, K, _zrow, 0)

    def _zhist(i, _):
        hist_v[pl.ds(i * 16, 16)] = zero16
        return 0
    lax.fori_loop(0, NH // 16, _zhist, 0)

    nzch = ZROWS // K
    def _zacc(k, _):
        ch = k * NS + s

        @pl.when(ch < nzch)
        def _():
            pltpu.sync_copy(rows2.at[0], zsh.at[pl.ds(ch * K, K)])
        return 0
    lax.fori_loop(0, -(-nzch // NS), _zacc, 0)

    pltpu.sync_copy(src_hbm.at[s], src_v.at[pl.ds(0, EPW)])
    pltpu.sync_copy(dst_hbm.at[s], dst_v.at[pl.ds(0, EPW)])

    def _cmp(i, off):
        vs = src_v[pl.ds(i * 16, 16)]
        vd = dst_v[pl.ds(i * 16, 16)]
        local = vd - base
        ok = (local >= 0) & (local < NH)
        plsc.store_compressed(src_v.at[pl.ds(off, 16)], vs, mask=ok)
        plsc.store_compressed(dst_v.at[pl.ds(off, 16)], local, mask=ok)
        return off + jnp.max(plsc.all_reduce_population_count(ok))
    cnt = lax.fori_loop(0, EPW // 16, _cmp, jnp.int32(0))

    nblk = (cnt + K - 1) // K
    padn = nblk * K - cnt
    for t in range(K // 16):
        @pl.when(t * 16 < padn)
        def _(t=t):
            src_v[pl.ds(cnt + t * 16, 16)] = N + lane + 16 * t
            dst_v[pl.ds(cnt + t * 16, 16)] = NH + lane + 16 * t

    def _rl(r, _):
        def _rc(t, _):
            dst2_v[r, pl.ds(t * 16, 16)] = dst_v[pl.ds(r * K + t * 16, 16)]
            return 0
        lax.fori_loop(0, K // 16, _rc, 0)
        return 0
    lax.fori_loop(0, nblk, _rl, 0)

    plsc.subcore_barrier()

    def _blk(j, _):
        cp = pltpu.async_copy(m_hbm.at[src_v.at[pl.ds(j * K, K)]],
                              rows2.at[0], sems.at[0])

        def _hist(t, _):
            idx = dst2_v[j, pl.ds(t * 16, 16)]
            plsc.addupdate_scatter(hist_v, [idx], one16, mask=idx < NH)
            return 0
        lax.fori_loop(0, K // 16, _hist, 0)

        cp.wait()
        pltpu.sync_copy(rows2.at[0], zsh.at[dst2_v.at[j]], add=True)
        return 0
    lax.fori_loop(0, nblk, _blk, 0)

    plsc.subcore_barrier()

    sl = pl.ds(s * RPC, RPC)
    pltpu.sync_copy(zsh.at[sl], z_out.at[pl.ds(base + s * RPC, RPC)])
    pltpu.sync_copy(hist_v, deg_out.at[c, s])


def _sc_edge_pass(m_pad, src3, dst3):
    mesh = plsc.VectorSubcoreMesh(
        core_axis_name="c", subcore_axis_name="s",
        num_cores=NC, num_subcores=NS,
    )
    return pl.kernel(
        _sc_edge_body,
        out_type=(
            jax.ShapeDtypeStruct((N_PAD, D), jnp.float32),
            jax.ShapeDtypeStruct((NC, NS, NH), jnp.float32),
        ),
        mesh=mesh,
        scratch_types=[
            pltpu.VMEM((EPW + 16,), jnp.int32),
            pltpu.VMEM((EPW + 16,), jnp.int32),
            pltpu.VMEM((NB, K), jnp.int32),
            pltpu.VMEM((1, K, D), jnp.float32),
            pltpu.VMEM((NH,), jnp.float32),
            pltpu.VMEM_SHARED((ZROWS, D), jnp.float32),
            pltpu.SemaphoreType.DMA((2,)),
        ],
        compiler_params=pltpu.CompilerParams(needs_layout_passes=False),
    )(m_pad, src3, dst3)


def _update_body(z_ref, d_ref, x_ref, w_ref, b_ref, o_ref):
    ones_cols = jnp.ones((NS, D), jnp.float32)
    deg_mat = lax.dot_general(
        d_ref[0], ones_cols, (((0,), (0,)), ((), ())),
        preferred_element_type=jnp.float32,
    )
    z = z_ref[:] / jnp.maximum(deg_mat, 1.0)
    o_ref[:] = jax.nn.relu(
        jnp.dot(z, w_ref[:], preferred_element_type=jnp.float32) + b_ref[:]
    ) + x_ref[:]


def _node_update(z, deg, x, Wu, bu2):
    return pl.pallas_call(
        _update_body,
        grid=(N_PAD // TCB,),
        in_specs=[
            pl.BlockSpec((TCB, D), lambda i: (i, 0)),
            pl.BlockSpec(
                (1, NS, TCB),
                lambda i: (i // (NH // TCB), 0, i % (NH // TCB)),
            ),
            pl.BlockSpec((TCB, D), lambda i: (i, 0)),
            pl.BlockSpec((D, D), lambda i: (0, 0)),
            pl.BlockSpec((1, D), lambda i: (0, 0)),
        ],
        out_specs=pl.BlockSpec((TCB, D), lambda i: (i, 0)),
        out_shape=jax.ShapeDtypeStruct((N, D), jnp.float32),
    )(z, deg, x, Wu, bu2)


def kernel(x, edge_index, Wp, bp, Wu, bu):
    x = x.astype(jnp.float32)

    src = edge_index[0].astype(jnp.int32)
    dst = edge_index[1].astype(jnp.int32)
    pad = E_PAD - E
    fill = N + (jnp.arange(pad, dtype=jnp.int32) % K)
    src3 = jnp.concatenate([src, fill]).reshape(NS, EPW)
    dst3 = jnp.concatenate([dst, fill]).reshape(NS, EPW)

    m_pad = _msg_precompute(x, Wp, bp.reshape(1, D))
    z, deg = _sc_edge_pass(m_pad, src3, dst3)
    return _node_update(z, deg, x, Wu, bu.reshape(1, D))


# speedup vs baseline: 8.7464x; 1.0471x over previous
import jax
import jax.numpy as jnp
from jax import lax
from jax.experimental import pallas as pl
from jax.experimental.pallas import tpu as pltpu
from jax.experimental.pallas import tpu_sc as plsc

N = 10000
D = 128
E = 320000

NC = 2
NS = 16
K = 128
_BLK = -(-(E // NS) // K)
NB = _BLK + (_BLK % 2)
EPW = NB * K
E_PAD = NS * EPW
NBH = NB // 2
EPH = NBH * K
N_PAD = 10240
NH = N_PAD // NC
ZROWS = NH + K
RPC = NH // NS
TCB = 256


def _mlp_relu_body(x_ref, w_ref, b_ref, o_ref):
    o_ref[:] = jax.nn.relu(
        jnp.dot(x_ref[:], w_ref[:], preferred_element_type=jnp.float32)
        + b_ref[:]
    )


def _msg_precompute(x, Wp, bp2):
    return pl.pallas_call(
        _mlp_relu_body,
        grid=(N_PAD // TCB,),
        in_specs=[
            pl.BlockSpec((TCB, D), lambda i: (i, 0)),
            pl.BlockSpec((D, D), lambda i: (0, 0)),
            pl.BlockSpec((1, D), lambda i: (0, 0)),
        ],
        out_specs=pl.BlockSpec((TCB, D), lambda i: (i, 0)),
        out_shape=jax.ShapeDtypeStruct((N_PAD, D), jnp.float32),
    )(x, Wp, bp2)


def _sc_edge_body(m_hbm, src_hbm, dst_hbm, z_out, deg_out,
                  src_v, dst_v, dst2_v, rows2, hist_v, zsh, sems):
    c = lax.axis_index("c")
    s = lax.axis_index("s")

    zero16 = jnp.zeros((16,), jnp.float32)
    one16 = jnp.ones((16,), jnp.float32)
    lane = lax.iota(jnp.int32, 16)
    base = c * NH

    def _zrow(i, _):
        def _zcol(j, _):
            rows2[0, i, pl.ds(j * 16, 16)] = zero16
            return 0
        lax.fori_loop(0, D // 16, _zcol, 0)
        return 0
    lax.fori_loop(0, K, _zrow, 0)

    def _zhist(i, _):
        hist_v[pl.ds(i * 16, 16)] = zero16
        return 0
    lax.fori_loop(0, NH // 16, _zhist, 0)

    nzch = ZROWS // K
    def _zacc(k, _):
        ch = k * NS + s

        @pl.when(ch < nzch)
        def _():
            pltpu.sync_copy(rows2.at[0], zsh.at[pl.ds(ch * K, K)])
        return 0
    lax.fori_loop(0, -(-nzch // NS), _zacc, 0)

    plsc.subcore_barrier()

    def _half(h, _):
        hb = s * EPW + h * EPH

        pltpu.sync_copy(src_hbm.at[pl.ds(hb, EPH)], src_v.at[pl.ds(0, EPH)])
        pltpu.sync_copy(dst_hbm.at[pl.ds(hb, EPH)], dst_v.at[pl.ds(0, EPH)])

        def _cmp(i, off):
            vs = src_v[pl.ds(i * 16, 16)]
            vd = dst_v[pl.ds(i * 16, 16)]
            local = vd - base
            ok = (local >= 0) & (local < NH)
            plsc.store_compressed(src_v.at[pl.ds(off, 16)], vs, mask=ok)
            plsc.store_compressed(dst_v.at[pl.ds(off, 16)], local, mask=ok)
            return off + jnp.max(plsc.all_reduce_population_count(ok))
        cnt = lax.fori_loop(0, EPH // 16, _cmp, jnp.int32(0))

        nblk = (cnt + K - 1) // K
        padn = nblk * K - cnt
        for t in range(K // 16):
            @pl.when(t * 16 < padn)
            def _(t=t):
                src_v[pl.ds(cnt + t * 16, 16)] = N + lane + 16 * t
                dst_v[pl.ds(cnt + t * 16, 16)] = NH + lane + 16 * t

        def _rl(r, _):
            def _rc(t, _):
                dst2_v[r, pl.ds(t * 16, 16)] = \
                    dst_v[pl.ds(r * K + t * 16, 16)]
                return 0
            lax.fori_loop(0, K // 16, _rc, 0)
            return 0
        lax.fori_loop(0, nblk, _rl, 0)

        def _blk(i, _):
            ja = 2 * i
            jb = 2 * i + 1
            da = pltpu.async_copy(m_hbm.at[src_v.at[pl.ds(ja * K, K)]],
                                  rows2.at[0], sems.at[0])
            db = pltpu.async_copy(m_hbm.at[src_v.at[pl.ds(jb * K, K)]],
                                  rows2.at[1], sems.at[1])

            def _hist(t, _):
                idx = dst2_v[ja + t // 8, pl.ds((t % 8) * 16, 16)]
                plsc.addupdate_scatter(hist_v, [idx], one16,
                                       mask=idx < NH)
                return 0
            lax.fori_loop(0, 2 * (K // 16), _hist, 0)

            da.wait()
            pltpu.sync_copy(rows2.at[0], zsh.at[dst2_v.at[ja]], add=True)
            db.wait()
            pltpu.sync_copy(rows2.at[1], zsh.at[dst2_v.at[jb]], add=True)
            return 0
        lax.fori_loop(0, nblk // 2, _blk, 0)

        @pl.when(nblk % 2 == 1)
        def _():
            j = nblk - 1
            cp = pltpu.async_copy(m_hbm.at[src_v.at[pl.ds(j * K, K)]],
                                  rows2.at[0], sems.at[0])

            def _hist(t, _):
                idx = dst2_v[j, pl.ds(t * 16, 16)]
                plsc.addupdate_scatter(hist_v, [idx], one16,
                                       mask=idx < NH)
                return 0
            lax.fori_loop(0, K // 16, _hist, 0)

            cp.wait()
            pltpu.sync_copy(rows2.at[0], zsh.at[dst2_v.at[j]], add=True)
        return 0
    lax.fori_loop(0, 2, _half, 0)

    plsc.subcore_barrier()

    sl = pl.ds(s * RPC, RPC)
    pltpu.sync_copy(zsh.at[sl], z_out.at[pl.ds(base + s * RPC, RPC)])
    pltpu.sync_copy(hist_v, deg_out.at[c, s])


def _sc_edge_pass(m_pad, src3, dst3):
    mesh = plsc.VectorSubcoreMesh(
        core_axis_name="c", subcore_axis_name="s",
        num_cores=NC, num_subcores=NS,
    )
    return pl.kernel(
        _sc_edge_body,
        out_type=(
            jax.ShapeDtypeStruct((N_PAD, D), jnp.float32),
            jax.ShapeDtypeStruct((NC, NS, NH), jnp.float32),
        ),
        mesh=mesh,
        scratch_types=[
            pltpu.VMEM((EPH + 16,), jnp.int32),
            pltpu.VMEM((EPH + 16,), jnp.int32),
            pltpu.VMEM((NBH, K), jnp.int32),
            pltpu.VMEM((2, K, D), jnp.float32),
            pltpu.VMEM((NH,), jnp.float32),
            pltpu.VMEM_SHARED((ZROWS, D), jnp.float32),
            pltpu.SemaphoreType.DMA((2,)),
        ],
        compiler_params=pltpu.CompilerParams(needs_layout_passes=False),
    )(m_pad, src3, dst3)


def _update_body(z_ref, d_ref, x_ref, w_ref, b_ref, o_ref):
    ones_cols = jnp.ones((NS, D), jnp.float32)
    deg_mat = lax.dot_general(
        d_ref[0], ones_cols, (((0,), (0,)), ((), ())),
        preferred_element_type=jnp.float32,
    )
    z = z_ref[:] / jnp.maximum(deg_mat, 1.0)
    o_ref[:] = jax.nn.relu(
        jnp.dot(z, w_ref[:], preferred_element_type=jnp.float32) + b_ref[:]
    ) + x_ref[:]


def _node_update(z, deg, x, Wu, bu2):
    return pl.pallas_call(
        _update_body,
        grid=(N_PAD // TCB,),
        in_specs=[
            pl.BlockSpec((TCB, D), lambda i: (i, 0)),
            pl.BlockSpec(
                (1, NS, TCB),
                lambda i: (i // (NH // TCB), 0, i % (NH // TCB)),
            ),
            pl.BlockSpec((TCB, D), lambda i: (i, 0)),
            pl.BlockSpec((D, D), lambda i: (0, 0)),
            pl.BlockSpec((1, D), lambda i: (0, 0)),
        ],
        out_specs=pl.BlockSpec((TCB, D), lambda i: (i, 0)),
        out_shape=jax.ShapeDtypeStruct((N, D), jnp.float32),
    )(z, deg, x, Wu, bu2)


def kernel(x, edge_index, Wp, bp, Wu, bu):
    x = x.astype(jnp.float32)

    src = edge_index[0].astype(jnp.int32)
    dst = edge_index[1].astype(jnp.int32)
    pad = E_PAD - E
    fill = N + (jnp.arange(pad, dtype=jnp.int32) % K)
    src3 = jnp.concatenate([src, fill])
    dst3 = jnp.concatenate([dst, fill])

    m_pad = _msg_precompute(x, Wp, bp.reshape(1, D))
    z, deg = _sc_edge_pass(m_pad, src3, dst3)
    return _node_update(z, deg, x, Wu, bu.reshape(1, D))


# speedup vs baseline: 10.0916x; 1.1538x over previous
import jax
import jax.numpy as jnp
from jax import lax
from jax.experimental import pallas as pl
from jax.experimental.pallas import tpu as pltpu
from jax.experimental.pallas import tpu_sc as plsc

N = 10000
D = 128
E = 320000

NC = 2
NS = 16
K = 128
_BLK = -(-(E // NS) // K)
NB = _BLK + (_BLK % 2)
EPW = NB * K
E_PAD = NS * EPW
NBH = NB // 2
EPH = NBH * K
N_PAD = 10240
NH = N_PAD // NC
ZROWS = NH + K
RPC = NH // NS
TCB = 1024


def _mlp_relu_body(x_ref, w_ref, b_ref, o_ref):
    o_ref[:] = jax.nn.relu(
        jnp.dot(x_ref[:], w_ref[:], preferred_element_type=jnp.float32)
        + b_ref[:]
    )


def _msg_precompute(x, Wp, bp2):
    return pl.pallas_call(
        _mlp_relu_body,
        grid=(N_PAD // TCB,),
        in_specs=[
            pl.BlockSpec((TCB, D), lambda i: (i, 0)),
            pl.BlockSpec((D, D), lambda i: (0, 0)),
            pl.BlockSpec((1, D), lambda i: (0, 0)),
        ],
        out_specs=pl.BlockSpec((TCB, D), lambda i: (i, 0)),
        out_shape=jax.ShapeDtypeStruct((N_PAD, D), jnp.float32),
    )(x, Wp, bp2)


def _sc_edge_body(m_hbm, src_hbm, dst_hbm, z_out, deg_out,
                  src_v, dst_v, dst2_v, rows2, hist_v, zsh, sems):
    c = lax.axis_index("c")
    s = lax.axis_index("s")

    zero16 = jnp.zeros((16,), jnp.float32)
    one16 = jnp.ones((16,), jnp.float32)
    lane = lax.iota(jnp.int32, 16)
    base = c * NH

    def _zrow(i, _):
        def _zcol(j, _):
            rows2[0, i, pl.ds(j * 16, 16)] = zero16
            return 0
        lax.fori_loop(0, D // 16, _zcol, 0)
        return 0
    lax.fori_loop(0, K, _zrow, 0)

    def _zhist(i, _):
        hist_v[pl.ds(i * 16, 16)] = zero16
        return 0
    lax.fori_loop(0, NH // 16, _zhist, 0)

    nzch = ZROWS // K
    def _zacc(k, _):
        ch = k * NS + s

        @pl.when(ch < nzch)
        def _():
            pltpu.sync_copy(rows2.at[0], zsh.at[pl.ds(ch * K, K)])
        return 0
    lax.fori_loop(0, -(-nzch // NS), _zacc, 0)

    plsc.subcore_barrier()

    def _half(h, _):
        hb = s * EPW + h * EPH

        pltpu.sync_copy(src_hbm.at[pl.ds(hb, EPH)], src_v.at[pl.ds(0, EPH)])
        pltpu.sync_copy(dst_hbm.at[pl.ds(hb, EPH)], dst_v.at[pl.ds(0, EPH)])

        def _cmp(i, off):
            vs = src_v[pl.ds(i * 16, 16)]
            vd = dst_v[pl.ds(i * 16, 16)]
            local = vd - base
            ok = (local >= 0) & (local < NH)
            plsc.store_compressed(src_v.at[pl.ds(off, 16)], vs, mask=ok)
            plsc.store_compressed(dst_v.at[pl.ds(off, 16)], local, mask=ok)
            return off + jnp.max(plsc.all_reduce_population_count(ok))
        cnt = lax.fori_loop(0, EPH // 16, _cmp, jnp.int32(0))

        nblk = (cnt + K - 1) // K
        padn = nblk * K - cnt
        for t in range(K // 16):
            @pl.when(t * 16 < padn)
            def _(t=t):
                src_v[pl.ds(cnt + t * 16, 16)] = N + lane + 16 * t
                dst_v[pl.ds(cnt + t * 16, 16)] = NH + lane + 16 * t

        def _rl(r, _):
            def _rc(t, _):
                dst2_v[r, pl.ds(t * 16, 16)] = \
                    dst_v[pl.ds(r * K + t * 16, 16)]
                return 0
            lax.fori_loop(0, K // 16, _rc, 0)
            return 0
        lax.fori_loop(0, nblk, _rl, 0)

        def _blk(i, _):
            ja = 2 * i
            jb = 2 * i + 1
            da = pltpu.async_copy(m_hbm.at[src_v.at[pl.ds(ja * K, K)]],
                                  rows2.at[0], sems.at[0])
            db = pltpu.async_copy(m_hbm.at[src_v.at[pl.ds(jb * K, K)]],
                                  rows2.at[1], sems.at[1])

            def _hist(t, _):
                idx = dst2_v[ja + t // 8, pl.ds((t % 8) * 16, 16)]
                plsc.addupdate_scatter(hist_v, [idx], one16,
                                       mask=idx < NH)
                return 0
            lax.fori_loop(0, 2 * (K // 16), _hist, 0)

            da.wait()
            pltpu.sync_copy(rows2.at[0], zsh.at[dst2_v.at[ja]], add=True)
            db.wait()
            pltpu.sync_copy(rows2.at[1], zsh.at[dst2_v.at[jb]], add=True)
            return 0
        lax.fori_loop(0, nblk // 2, _blk, 0)

        @pl.when(nblk % 2 == 1)
        def _():
            j = nblk - 1
            cp = pltpu.async_copy(m_hbm.at[src_v.at[pl.ds(j * K, K)]],
                                  rows2.at[0], sems.at[0])

            def _hist(t, _):
                idx = dst2_v[j, pl.ds(t * 16, 16)]
                plsc.addupdate_scatter(hist_v, [idx], one16,
                                       mask=idx < NH)
                return 0
            lax.fori_loop(0, K // 16, _hist, 0)

            cp.wait()
            pltpu.sync_copy(rows2.at[0], zsh.at[dst2_v.at[j]], add=True)
        return 0
    lax.fori_loop(0, 2, _half, 0)

    plsc.subcore_barrier()

    sl = pl.ds(s * RPC, RPC)
    pltpu.sync_copy(zsh.at[sl], z_out.at[pl.ds(base + s * RPC, RPC)])
    pltpu.sync_copy(hist_v, deg_out.at[c, s])


def _sc_edge_pass(m_pad, src3, dst3):
    mesh = plsc.VectorSubcoreMesh(
        core_axis_name="c", subcore_axis_name="s",
        num_cores=NC, num_subcores=NS,
    )
    return pl.kernel(
        _sc_edge_body,
        out_type=(
            jax.ShapeDtypeStruct((N_PAD, D), jnp.float32),
            jax.ShapeDtypeStruct((NC, NS, NH), jnp.float32),
        ),
        mesh=mesh,
        scratch_types=[
            pltpu.VMEM((EPH + 16,), jnp.int32),
            pltpu.VMEM((EPH + 16,), jnp.int32),
            pltpu.VMEM((NBH, K), jnp.int32),
            pltpu.VMEM((2, K, D), jnp.float32),
            pltpu.VMEM((NH,), jnp.float32),
            pltpu.VMEM_SHARED((ZROWS, D), jnp.float32),
            pltpu.SemaphoreType.DMA((2,)),
        ],
        compiler_params=pltpu.CompilerParams(needs_layout_passes=False),
    )(m_pad, src3, dst3)


def _update_body(z_ref, d_ref, x_ref, w_ref, b_ref, o_ref):
    ones_cols = jnp.ones((NS, D), jnp.float32)
    deg_mat = lax.dot_general(
        d_ref[0], ones_cols, (((0,), (0,)), ((), ())),
        preferred_element_type=jnp.float32,
    )
    z = z_ref[:] / jnp.maximum(deg_mat, 1.0)
    o_ref[:] = jax.nn.relu(
        jnp.dot(z, w_ref[:], preferred_element_type=jnp.float32) + b_ref[:]
    ) + x_ref[:]


def _node_update(z, deg, x, Wu, bu2):
    return pl.pallas_call(
        _update_body,
        grid=(N_PAD // TCB,),
        in_specs=[
            pl.BlockSpec((TCB, D), lambda i: (i, 0)),
            pl.BlockSpec(
                (1, NS, TCB),
                lambda i: (i // (NH // TCB), 0, i % (NH // TCB)),
            ),
            pl.BlockSpec((TCB, D), lambda i: (i, 0)),
            pl.BlockSpec((D, D), lambda i: (0, 0)),
            pl.BlockSpec((1, D), lambda i: (0, 0)),
        ],
        out_specs=pl.BlockSpec((TCB, D), lambda i: (i, 0)),
        out_shape=jax.ShapeDtypeStruct((N, D), jnp.float32),
    )(z, deg, x, Wu, bu2)


def kernel(x, edge_index, Wp, bp, Wu, bu):
    x = x.astype(jnp.float32)

    src = edge_index[0].astype(jnp.int32)
    dst = edge_index[1].astype(jnp.int32)
    pad = E_PAD - E
    fill = N + (jnp.arange(pad, dtype=jnp.int32) % K)
    src3 = jnp.concatenate([src, fill])
    dst3 = jnp.concatenate([dst, fill])

    m_pad = _msg_precompute(x, Wp, bp.reshape(1, D))
    z, deg = _sc_edge_pass(m_pad, src3, dst3)
    return _node_update(z, deg, x, Wu, bu.reshape(1, D))


# speedup vs baseline: 11.3512x; 1.1248x over previous
import jax
import jax.numpy as jnp
from jax import lax
from jax.experimental import pallas as pl
from jax.experimental.pallas import tpu as pltpu
from jax.experimental.pallas import tpu_sc as plsc

N = 10000
D = 128
E = 320000

NC = 2
NS = 16
K = 128
_BLK = -(-(E // NS) // K)
NB = _BLK + (_BLK % 2)
EPW = NB * K
E_PAD = NS * EPW
NBH = NB // 2
EPH = NBH * K
N_PAD = 10240
NH = N_PAD // NC
ZROWS = NH + K
RPC = NH // NS
TCB = 1024


def _mlp_relu_body(x_ref, w_ref, b_ref, o_ref):
    o_ref[:] = jax.nn.relu(
        jnp.dot(x_ref[:], w_ref[:], preferred_element_type=jnp.float32)
        + b_ref[:]
    )


def _msg_precompute(x, Wp, bp2):
    return pl.pallas_call(
        _mlp_relu_body,
        grid=(N_PAD // TCB,),
        in_specs=[
            pl.BlockSpec((TCB, D), lambda i: (i, 0)),
            pl.BlockSpec((D, D), lambda i: (0, 0)),
            pl.BlockSpec((1, D), lambda i: (0, 0)),
        ],
        out_specs=pl.BlockSpec((TCB, D), lambda i: (i, 0)),
        out_shape=jax.ShapeDtypeStruct((N_PAD, D), jnp.float32),
    )(x, Wp, bp2)


def _sc_edge_body(m_hbm, ei_hbm, z_out, deg_out,
                  src_v, dst_v, dst2_v, rows2, hist_v, zsh, sems):
    c = lax.axis_index("c")
    s = lax.axis_index("s")

    zero16 = jnp.zeros((16,), jnp.float32)
    one16 = jnp.ones((16,), jnp.float32)
    lane = lax.iota(jnp.int32, 16)
    base = c * NH

    def _zrow(i, _):
        def _zcol(j, _):
            rows2[0, i, pl.ds(j * 16, 16)] = zero16
            return 0
        lax.fori_loop(0, D // 16, _zcol, 0)
        return 0
    lax.fori_loop(0, K, _zrow, 0)

    def _zhist(i, _):
        hist_v[pl.ds(i * 16, 16)] = zero16
        return 0
    lax.fori_loop(0, NH // 16, _zhist, 0)

    nzch = ZROWS // K
    def _zacc(k, _):
        ch = k * NS + s

        @pl.when(ch < nzch)
        def _():
            pltpu.sync_copy(rows2.at[0], zsh.at[pl.ds(ch * K, K)])
        return 0
    lax.fori_loop(0, -(-nzch // NS), _zacc, 0)

    plsc.subcore_barrier()

    def _half(h, _):
        hb = s * EPW + h * EPH

        start = jnp.minimum(hb, E - EPH)
        dup = hb - start
        pltpu.sync_copy(ei_hbm.at[0, pl.ds(start, EPH)],
                        src_v.at[pl.ds(0, EPH)])
        pltpu.sync_copy(ei_hbm.at[1, pl.ds(start, EPH)],
                        dst_v.at[pl.ds(0, EPH)])

        def _cmp(i, off):
            vs = src_v[pl.ds(i * 16, 16)]
            vd = dst_v[pl.ds(i * 16, 16)]
            local = vd - base
            ok = (local >= 0) & (local < NH) & (i * 16 + lane >= dup)
            plsc.store_compressed(src_v.at[pl.ds(off, 16)], vs, mask=ok)
            plsc.store_compressed(dst_v.at[pl.ds(off, 16)], local, mask=ok)
            return off + jnp.max(plsc.all_reduce_population_count(ok))
        cnt = lax.fori_loop(0, EPH // 16, _cmp, jnp.int32(0))

        nblk = (cnt + K - 1) // K
        padn = nblk * K - cnt
        for t in range(K // 16):
            @pl.when(t * 16 < padn)
            def _(t=t):
                src_v[pl.ds(cnt + t * 16, 16)] = N + lane + 16 * t
                dst_v[pl.ds(cnt + t * 16, 16)] = NH + lane + 16 * t

        def _rl(r, _):
            def _rc(t, _):
                dst2_v[r, pl.ds(t * 16, 16)] = \
                    dst_v[pl.ds(r * K + t * 16, 16)]
                return 0
            lax.fori_loop(0, K // 16, _rc, 0)
            return 0
        lax.fori_loop(0, nblk, _rl, 0)

        def _blk(i, _):
            ja = 2 * i
            jb = 2 * i + 1
            da = pltpu.async_copy(m_hbm.at[src_v.at[pl.ds(ja * K, K)]],
                                  rows2.at[0], sems.at[0])
            db = pltpu.async_copy(m_hbm.at[src_v.at[pl.ds(jb * K, K)]],
                                  rows2.at[1], sems.at[1])

            def _hist(t, _):
                idx = dst2_v[ja + t // 8, pl.ds((t % 8) * 16, 16)]
                plsc.addupdate_scatter(hist_v, [idx], one16,
                                       mask=idx < NH)
                return 0
            lax.fori_loop(0, 2 * (K // 16), _hist, 0)

            da.wait()
            pltpu.sync_copy(rows2.at[0], zsh.at[dst2_v.at[ja]], add=True)
            db.wait()
            pltpu.sync_copy(rows2.at[1], zsh.at[dst2_v.at[jb]], add=True)
            return 0
        lax.fori_loop(0, nblk // 2, _blk, 0)

        @pl.when(nblk % 2 == 1)
        def _():
            j = nblk - 1
            cp = pltpu.async_copy(m_hbm.at[src_v.at[pl.ds(j * K, K)]],
                                  rows2.at[0], sems.at[0])

            def _hist(t, _):
                idx = dst2_v[j, pl.ds(t * 16, 16)]
                plsc.addupdate_scatter(hist_v, [idx], one16,
                                       mask=idx < NH)
                return 0
            lax.fori_loop(0, K // 16, _hist, 0)

            cp.wait()
            pltpu.sync_copy(rows2.at[0], zsh.at[dst2_v.at[j]], add=True)
        return 0
    lax.fori_loop(0, 2, _half, 0)

    plsc.subcore_barrier()

    sl = pl.ds(s * RPC, RPC)
    pltpu.sync_copy(zsh.at[sl], z_out.at[pl.ds(base + s * RPC, RPC)])
    pltpu.sync_copy(hist_v, deg_out.at[c, s])


def _sc_edge_pass(m_pad, ei):
    mesh = plsc.VectorSubcoreMesh(
        core_axis_name="c", subcore_axis_name="s",
        num_cores=NC, num_subcores=NS,
    )
    return pl.kernel(
        _sc_edge_body,
        out_type=(
            jax.ShapeDtypeStruct((N_PAD, D), jnp.float32),
            jax.ShapeDtypeStruct((NC, NS, NH), jnp.float32),
        ),
        mesh=mesh,
        scratch_types=[
            pltpu.VMEM((EPH + 16,), jnp.int32),
            pltpu.VMEM((EPH + 16,), jnp.int32),
            pltpu.VMEM((NBH, K), jnp.int32),
            pltpu.VMEM((2, K, D), jnp.float32),
            pltpu.VMEM((NH,), jnp.float32),
            pltpu.VMEM_SHARED((ZROWS, D), jnp.float32),
            pltpu.SemaphoreType.DMA((2,)),
        ],
        compiler_params=pltpu.CompilerParams(needs_layout_passes=False),
    )(m_pad, ei)


def _update_body(z_ref, d_ref, x_ref, w_ref, b_ref, o_ref):
    ones_cols = jnp.ones((NS, D), jnp.float32)
    deg_mat = lax.dot_general(
        d_ref[0], ones_cols, (((0,), (0,)), ((), ())),
        preferred_element_type=jnp.float32,
    )
    z = z_ref[:] / jnp.maximum(deg_mat, 1.0)
    o_ref[:] = jax.nn.relu(
        jnp.dot(z, w_ref[:], preferred_element_type=jnp.float32) + b_ref[:]
    ) + x_ref[:]


def _node_update(z, deg, x, Wu, bu2):
    return pl.pallas_call(
        _update_body,
        grid=(N_PAD // TCB,),
        in_specs=[
            pl.BlockSpec((TCB, D), lambda i: (i, 0)),
            pl.BlockSpec(
                (1, NS, TCB),
                lambda i: (i // (NH // TCB), 0, i % (NH // TCB)),
            ),
            pl.BlockSpec((TCB, D), lambda i: (i, 0)),
            pl.BlockSpec((D, D), lambda i: (0, 0)),
            pl.BlockSpec((1, D), lambda i: (0, 0)),
        ],
        out_specs=pl.BlockSpec((TCB, D), lambda i: (i, 0)),
        out_shape=jax.ShapeDtypeStruct((N, D), jnp.float32),
    )(z, deg, x, Wu, bu2)


def kernel(x, edge_index, Wp, bp, Wu, bu):
    x = x.astype(jnp.float32)
    ei = edge_index.astype(jnp.int32)

    m_pad = _msg_precompute(x, Wp, bp.reshape(1, D))
    z, deg = _sc_edge_pass(m_pad, ei)
    return _node_update(z, deg, x, Wu, bu.reshape(1, D))


# speedup vs baseline: 11.3969x; 1.0040x over previous
import jax
import jax.numpy as jnp
from jax import lax
from jax.experimental import pallas as pl
from jax.experimental.pallas import tpu as pltpu
from jax.experimental.pallas import tpu_sc as plsc

N = 10000
D = 128
E = 320000

NC = 2
NS = 16
K = 128
_BLK = -(-(E // NS) // K)
NB = _BLK + (_BLK % 2)
EPW = NB * K
E_PAD = NS * EPW
NBH = NB // 2
EPH = NBH * K
N_PAD = 10240
NH = N_PAD // NC
ZROWS = NH + K
RPC = NH // NS
TCB = 1024


def _mlp_relu_body(x_ref, w_ref, b_ref, o_ref):
    o_ref[:] = jax.nn.relu(
        jnp.dot(x_ref[:], w_ref[:], preferred_element_type=jnp.float32)
        + b_ref[:]
    )


def _msg_precompute(x, Wp, bp2):
    return pl.pallas_call(
        _mlp_relu_body,
        grid=(N_PAD // TCB,),
        in_specs=[
            pl.BlockSpec((TCB, D), lambda i: (i, 0)),
            pl.BlockSpec((D, D), lambda i: (0, 0)),
            pl.BlockSpec((1, D), lambda i: (0, 0)),
        ],
        out_specs=pl.BlockSpec((TCB, D), lambda i: (i, 0)),
        out_shape=jax.ShapeDtypeStruct((N_PAD, D), jnp.float32),
    )(x, Wp, bp2)


def _sc_edge_body(m_hbm, ei_hbm, z_out, deg_out,
                  src_v, dst_v, dst2_v, rows2, hist_v, zsh, sems):
    c = lax.axis_index("c")
    s = lax.axis_index("s")

    zero16 = jnp.zeros((16,), jnp.float32)
    one16 = jnp.ones((16,), jnp.float32)
    lane = lax.iota(jnp.int32, 16)
    base = c * NH

    def _zrow(i, _):
        def _zcol(j, _):
            rows2[0, i, pl.ds(j * 16, 16)] = zero16
            return 0
        lax.fori_loop(0, D // 16, _zcol, 0)
        return 0
    lax.fori_loop(0, K, _zrow, 0)

    def _zhist(i, _):
        hist_v[pl.ds(i * 16, 16)] = zero16
        return 0
    lax.fori_loop(0, NH // 16, _zhist, 0)

    nzch = ZROWS // K
    def _zacc(k, _):
        ch = k * NS + s

        @pl.when(ch < nzch)
        def _():
            pltpu.sync_copy(rows2.at[0], zsh.at[pl.ds(ch * K, K)])
        return 0
    lax.fori_loop(0, -(-nzch // NS), _zacc, 0)

    plsc.subcore_barrier()

    def _half(h, _):
        hb = s * EPW + h * EPH

        start = jnp.minimum(hb, E - EPH)
        dup = hb - start
        pltpu.sync_copy(ei_hbm.at[0, pl.ds(start, EPH)],
                        src_v.at[pl.ds(0, EPH)])
        pltpu.sync_copy(ei_hbm.at[1, pl.ds(start, EPH)],
                        dst_v.at[pl.ds(0, EPH)])

        def _cmp(i, off):
            vs = src_v[pl.ds(i * 16, 16)]
            vd = dst_v[pl.ds(i * 16, 16)]
            local = vd - base
            ok = (local >= 0) & (local < NH) & (i * 16 + lane >= dup)
            plsc.store_compressed(src_v.at[pl.ds(off, 16)], vs, mask=ok)
            plsc.store_compressed(dst_v.at[pl.ds(off, 16)], local, mask=ok)
            return off + plsc.all_reduce_population_count(ok)[0]
        cnt = lax.fori_loop(0, EPH // 16, _cmp, jnp.int32(0), unroll=4)

        nblk = (cnt + K - 1) // K
        padn = nblk * K - cnt
        for t in range(K // 16):
            @pl.when(t * 16 < padn)
            def _(t=t):
                src_v[pl.ds(cnt + t * 16, 16)] = N + lane + 16 * t
                dst_v[pl.ds(cnt + t * 16, 16)] = NH + lane + 16 * t

        def _rl(r, _):
            def _rc(t, _):
                dst2_v[r, pl.ds(t * 16, 16)] = \
                    dst_v[pl.ds(r * K + t * 16, 16)]
                return 0
            lax.fori_loop(0, K // 16, _rc, 0)
            return 0
        lax.fori_loop(0, nblk, _rl, 0)

        def _blk(i, _):
            ja = 2 * i
            jb = 2 * i + 1
            da = pltpu.async_copy(m_hbm.at[src_v.at[pl.ds(ja * K, K)]],
                                  rows2.at[0], sems.at[0])
            db = pltpu.async_copy(m_hbm.at[src_v.at[pl.ds(jb * K, K)]],
                                  rows2.at[1], sems.at[1])

            def _hist(t, _):
                idx = dst2_v[ja + t // 8, pl.ds((t % 8) * 16, 16)]
                plsc.addupdate_scatter(hist_v, [idx], one16,
                                       mask=idx < NH)
                return 0
            lax.fori_loop(0, 2 * (K // 16), _hist, 0)

            da.wait()
            pltpu.sync_copy(rows2.at[0], zsh.at[dst2_v.at[ja]], add=True)
            db.wait()
            pltpu.sync_copy(rows2.at[1], zsh.at[dst2_v.at[jb]], add=True)
            return 0
        lax.fori_loop(0, nblk // 2, _blk, 0)

        @pl.when(nblk % 2 == 1)
        def _():
            j = nblk - 1
            cp = pltpu.async_copy(m_hbm.at[src_v.at[pl.ds(j * K, K)]],
                                  rows2.at[0], sems.at[0])

            def _hist(t, _):
                idx = dst2_v[j, pl.ds(t * 16, 16)]
                plsc.addupdate_scatter(hist_v, [idx], one16,
                                       mask=idx < NH)
                return 0
            lax.fori_loop(0, K // 16, _hist, 0)

            cp.wait()
            pltpu.sync_copy(rows2.at[0], zsh.at[dst2_v.at[j]], add=True)
        return 0
    lax.fori_loop(0, 2, _half, 0)

    plsc.subcore_barrier()

    sl = pl.ds(s * RPC, RPC)
    pltpu.sync_copy(zsh.at[sl], z_out.at[pl.ds(base + s * RPC, RPC)])
    pltpu.sync_copy(hist_v, deg_out.at[c, s])


def _sc_edge_pass(m_pad, ei):
    mesh = plsc.VectorSubcoreMesh(
        core_axis_name="c", subcore_axis_name="s",
        num_cores=NC, num_subcores=NS,
    )
    return pl.kernel(
        _sc_edge_body,
        out_type=(
            jax.ShapeDtypeStruct((N_PAD, D), jnp.float32),
            jax.ShapeDtypeStruct((NC, NS, NH), jnp.float32),
        ),
        mesh=mesh,
        scratch_types=[
            pltpu.VMEM((EPH + 16,), jnp.int32),
            pltpu.VMEM((EPH + 16,), jnp.int32),
            pltpu.VMEM((NBH, K), jnp.int32),
            pltpu.VMEM((2, K, D), jnp.float32),
            pltpu.VMEM((NH,), jnp.float32),
            pltpu.VMEM_SHARED((ZROWS, D), jnp.float32),
            pltpu.SemaphoreType.DMA((2,)),
        ],
        compiler_params=pltpu.CompilerParams(needs_layout_passes=False),
    )(m_pad, ei)


def _update_body(z_ref, d_ref, x_ref, w_ref, b_ref, o_ref):
    ones_cols = jnp.ones((NS, D), jnp.float32)
    deg_mat = lax.dot_general(
        d_ref[0], ones_cols, (((0,), (0,)), ((), ())),
        preferred_element_type=jnp.float32,
    )
    z = z_ref[:] / jnp.maximum(deg_mat, 1.0)
    o_ref[:] = jax.nn.relu(
        jnp.dot(z, w_ref[:], preferred_element_type=jnp.float32) + b_ref[:]
    ) + x_ref[:]


def _node_update(z, deg, x, Wu, bu2):
    return pl.pallas_call(
        _update_body,
        grid=(N_PAD // TCB,),
        in_specs=[
            pl.BlockSpec((TCB, D), lambda i: (i, 0)),
            pl.BlockSpec(
                (1, NS, TCB),
                lambda i: (i // (NH // TCB), 0, i % (NH // TCB)),
            ),
            pl.BlockSpec((TCB, D), lambda i: (i, 0)),
            pl.BlockSpec((D, D), lambda i: (0, 0)),
            pl.BlockSpec((1, D), lambda i: (0, 0)),
        ],
        out_specs=pl.BlockSpec((TCB, D), lambda i: (i, 0)),
        out_shape=jax.ShapeDtypeStruct((N, D), jnp.float32),
    )(z, deg, x, Wu, bu2)


def kernel(x, edge_index, Wp, bp, Wu, bu):
    x = x.astype(jnp.float32)
    ei = edge_index.astype(jnp.int32)

    m_pad = _msg_precompute(x, Wp, bp.reshape(1, D))
    z, deg = _sc_edge_pass(m_pad, ei)
    return _node_update(z, deg, x, Wu, bu.reshape(1, D))


# speedup vs baseline: 14.5827x; 1.2795x over previous
import jax
import jax.numpy as jnp
from jax import lax
from jax.experimental import pallas as pl
from jax.experimental.pallas import tpu as pltpu
from jax.experimental.pallas import tpu_sc as plsc

N = 10000
D = 128
E = 320000

NC = 2
NS = 16
K = 128
_BLK = -(-(E // NS) // K)
NB = _BLK + (_BLK % 2)
EPW = NB * K
E_PAD = NS * EPW
NBH = NB // 2
EPH = NBH * K
N_PAD = 10240
NH = N_PAD // NC
ZROWS = NH + K
RPC = NH // NS
TCB = 1024


def _mlp_relu_body(x_ref, w_ref, b_ref, o_ref):
    o_ref[:] = jax.nn.relu(
        jnp.dot(x_ref[:], w_ref[:], preferred_element_type=jnp.float32)
        + b_ref[:]
    )


def _msg_precompute(x, Wp, bp2):
    return pl.pallas_call(
        _mlp_relu_body,
        grid=(N_PAD // TCB,),
        in_specs=[
            pl.BlockSpec((TCB, D), lambda i: (i, 0)),
            pl.BlockSpec((D, D), lambda i: (0, 0)),
            pl.BlockSpec((1, D), lambda i: (0, 0)),
        ],
        out_specs=pl.BlockSpec((TCB, D), lambda i: (i, 0)),
        out_shape=jax.ShapeDtypeStruct((N_PAD, D), jnp.float32),
    )(x, Wp, bp2)


def _sc_edge_body(m_hbm, ei_hbm, z_out, deg_out,
                  src_v, dst_v, dst2_v, rows2, hist_v, zsh, sems):
    c = lax.axis_index("c")
    s = lax.axis_index("s")

    zero16 = jnp.zeros((16,), jnp.float32)
    one16 = jnp.ones((16,), jnp.float32)
    lane = lax.iota(jnp.int32, 16)
    base = c * NH

    def _zrow(i, _):
        def _zcol(j, _):
            rows2[0, i, pl.ds(j * 16, 16)] = zero16
            return 0
        lax.fori_loop(0, D // 16, _zcol, 0)
        return 0
    lax.fori_loop(0, K, _zrow, 0)

    def _zhist(i, _):
        hist_v[pl.ds(i * 16, 16)] = zero16
        return 0
    lax.fori_loop(0, NH // 16, _zhist, 0)

    nzch = ZROWS // K
    def _zacc(k, _):
        ch = k * NS + s

        @pl.when(ch < nzch)
        def _():
            pltpu.sync_copy(rows2.at[0], zsh.at[pl.ds(ch * K, K)])
        return 0
    lax.fori_loop(0, -(-nzch // NS), _zacc, 0)

    plsc.subcore_barrier()

    def _half(h, _):
        hb = s * EPW + h * EPH

        start = jnp.minimum(hb, E - EPH)
        dup = hb - start
        pltpu.sync_copy(ei_hbm.at[0, pl.ds(start, EPH)],
                        src_v.at[pl.ds(0, EPH)])
        pltpu.sync_copy(ei_hbm.at[1, pl.ds(start, EPH)],
                        dst_v.at[pl.ds(0, EPH)])

        def _cmp(i, off):
            vs = src_v[pl.ds(i * 16, 16)]
            vd = dst_v[pl.ds(i * 16, 16)]
            local = vd - base
            ok = (local >= 0) & (local < NH) & (i * 16 + lane >= dup)
            plsc.store_compressed(src_v.at[pl.ds(off, 16)], vs, mask=ok)
            plsc.store_compressed(dst_v.at[pl.ds(off, 16)], local, mask=ok)
            return off + plsc.all_reduce_population_count(ok)[0]
        cnt = lax.fori_loop(0, EPH // 16, _cmp, jnp.int32(0), unroll=4)

        nblk = (cnt + K - 1) // K
        padn = nblk * K - cnt
        for t in range(K // 16):
            @pl.when(t * 16 < padn)
            def _(t=t):
                src_v[pl.ds(cnt + t * 16, 16)] = N + lane + 16 * t
                dst_v[pl.ds(cnt + t * 16, 16)] = NH + lane + 16 * t

        def _rl(r, _):
            def _rc(t, _):
                dst2_v[r, pl.ds(t * 16, 16)] = \
                    dst_v[pl.ds(r * K + t * 16, 16)]
                return 0
            lax.fori_loop(0, K // 16, _rc, 0)
            return 0
        lax.fori_loop(0, nblk, _rl, 0)

        @pl.when(nblk > 0)
        def _():
            pltpu.async_copy(m_hbm.at[src_v.at[pl.ds(0, K)]],
                             rows2.at[0], sems.at[0])

        def _blk(j, _):
            p = lax.rem(j, 2)
            pn = 1 - p

            @pl.when(j + 1 < nblk)
            def _():
                pltpu.async_copy(
                    m_hbm.at[src_v.at[pl.ds((j + 1) * K, K)]],
                    rows2.at[pn], sems.at[pn])

            def _hist(t, _):
                idx = dst2_v[j, pl.ds(t * 16, 16)]
                plsc.addupdate_scatter(hist_v, [idx], one16,
                                       mask=idx < NH)
                return 0
            lax.fori_loop(0, K // 16, _hist, 0)

            pltpu.make_async_copy(m_hbm.at[src_v.at[pl.ds(j * K, K)]],
                                  rows2.at[p], sems.at[p]).wait()
            pltpu.sync_copy(rows2.at[p], zsh.at[dst2_v.at[j]], add=True)
            return 0
        lax.fori_loop(0, nblk, _blk, 0)
        return 0
    lax.fori_loop(0, 2, _half, 0)

    plsc.subcore_barrier()

    sl = pl.ds(s * RPC, RPC)
    pltpu.sync_copy(zsh.at[sl], z_out.at[pl.ds(base + s * RPC, RPC)])
    pltpu.sync_copy(hist_v, deg_out.at[c, s])


def _sc_edge_pass(m_pad, ei):
    mesh = plsc.VectorSubcoreMesh(
        core_axis_name="c", subcore_axis_name="s",
        num_cores=NC, num_subcores=NS,
    )
    return pl.kernel(
        _sc_edge_body,
        out_type=(
            jax.ShapeDtypeStruct((N_PAD, D), jnp.float32),
            jax.ShapeDtypeStruct((NC, NS, NH), jnp.float32),
        ),
        mesh=mesh,
        scratch_types=[
            pltpu.VMEM((EPH + 16,), jnp.int32),
            pltpu.VMEM((EPH + 16,), jnp.int32),
            pltpu.VMEM((NBH, K), jnp.int32),
            pltpu.VMEM((2, K, D), jnp.float32),
            pltpu.VMEM((NH,), jnp.float32),
            pltpu.VMEM_SHARED((ZROWS, D), jnp.float32),
            pltpu.SemaphoreType.DMA((2,)),
        ],
        compiler_params=pltpu.CompilerParams(needs_layout_passes=False),
    )(m_pad, ei)


def _update_body(z_ref, d_ref, x_ref, w_ref, b_ref, o_ref):
    ones_cols = jnp.ones((NS, D), jnp.float32)
    deg_mat = lax.dot_general(
        d_ref[0], ones_cols, (((0,), (0,)), ((), ())),
        preferred_element_type=jnp.float32,
    )
    z = z_ref[:] / jnp.maximum(deg_mat, 1.0)
    o_ref[:] = jax.nn.relu(
        jnp.dot(z, w_ref[:], preferred_element_type=jnp.float32) + b_ref[:]
    ) + x_ref[:]


def _node_update(z, deg, x, Wu, bu2):
    return pl.pallas_call(
        _update_body,
        grid=(N_PAD // TCB,),
        in_specs=[
            pl.BlockSpec((TCB, D), lambda i: (i, 0)),
            pl.BlockSpec(
                (1, NS, TCB),
                lambda i: (i // (NH // TCB), 0, i % (NH // TCB)),
            ),
            pl.BlockSpec((TCB, D), lambda i: (i, 0)),
            pl.BlockSpec((D, D), lambda i: (0, 0)),
            pl.BlockSpec((1, D), lambda i: (0, 0)),
        ],
        out_specs=pl.BlockSpec((TCB, D), lambda i: (i, 0)),
        out_shape=jax.ShapeDtypeStruct((N, D), jnp.float32),
    )(z, deg, x, Wu, bu2)


def kernel(x, edge_index, Wp, bp, Wu, bu):
    x = x.astype(jnp.float32)
    ei = edge_index.astype(jnp.int32)

    m_pad = _msg_precompute(x, Wp, bp.reshape(1, D))
    z, deg = _sc_edge_pass(m_pad, ei)
    return _node_update(z, deg, x, Wu, bu.reshape(1, D))


# speedup vs baseline: 14.5859x; 1.0002x over previous
import jax
import jax.numpy as jnp
from jax import lax
from jax.experimental import pallas as pl
from jax.experimental.pallas import tpu as pltpu
from jax.experimental.pallas import tpu_sc as plsc

N = 10000
D = 128
E = 320000

NC = 2
NS = 16
K = 128
_BLK = -(-(E // NS) // K)
NB = _BLK + (_BLK % 2)
EPW = NB * K
E_PAD = NS * EPW
NBH = NB // 2
EPH = NBH * K
N_PAD = 10240
NH = N_PAD // NC
ZROWS = NH + K
RPC = NH // NS
TCB = 1024


def _mlp_relu_body(x_ref, w_ref, b_ref, o_ref):
    o_ref[:] = jax.nn.relu(
        jnp.dot(x_ref[:], w_ref[:], preferred_element_type=jnp.float32)
        + b_ref[:]
    )


def _msg_precompute(x, Wp, bp2):
    return pl.pallas_call(
        _mlp_relu_body,
        grid=(N_PAD // TCB,),
        in_specs=[
            pl.BlockSpec((TCB, D), lambda i: (i, 0)),
            pl.BlockSpec((D, D), lambda i: (0, 0)),
            pl.BlockSpec((1, D), lambda i: (0, 0)),
        ],
        out_specs=pl.BlockSpec((TCB, D), lambda i: (i, 0)),
        out_shape=jax.ShapeDtypeStruct((N_PAD, D), jnp.float32),
    )(x, Wp, bp2)


def _sc_edge_body(m_hbm, ei_hbm, z_out, deg_out,
                  src_v, dst_v, dst2_v, rows2, hist_v, zsh, sems):
    c = lax.axis_index("c")
    s = lax.axis_index("s")

    zero16 = jnp.zeros((16,), jnp.float32)
    one16 = jnp.ones((16,), jnp.float32)
    lane = lax.iota(jnp.int32, 16)
    base = c * NH

    def _zrow(i, _):
        def _zcol(j, _):
            rows2[0, i, pl.ds(j * 16, 16)] = zero16
            return 0
        lax.fori_loop(0, D // 16, _zcol, 0)
        return 0
    lax.fori_loop(0, K, _zrow, 0)

    def _zhist(i, _):
        hist_v[pl.ds(i * 16, 16)] = zero16
        return 0
    lax.fori_loop(0, NH // 16, _zhist, 0)

    nzch = ZROWS // K
    def _zacc(k, _):
        ch = k * NS + s

        @pl.when(ch < nzch)
        def _():
            pltpu.sync_copy(rows2.at[0], zsh.at[pl.ds(ch * K, K)])
        return 0
    lax.fori_loop(0, -(-nzch // NS), _zacc, 0)

    plsc.subcore_barrier()

    def _half(h, _):
        hb = s * EPW + h * EPH

        start = jnp.minimum(hb, E - EPH)
        dup = hb - start
        pltpu.sync_copy(ei_hbm.at[0, pl.ds(start, EPH)],
                        src_v.at[pl.ds(0, EPH)])
        pltpu.sync_copy(ei_hbm.at[1, pl.ds(start, EPH)],
                        dst_v.at[pl.ds(0, EPH)])

        def _cmp(i, off):
            vs = src_v[pl.ds(i * 16, 16)]
            vd = dst_v[pl.ds(i * 16, 16)]
            local = vd - base
            ok = (local >= 0) & (local < NH) & (i * 16 + lane >= dup)
            plsc.store_compressed(src_v.at[pl.ds(off, 16)], vs, mask=ok)
            plsc.store_compressed(dst_v.at[pl.ds(off, 16)], local, mask=ok)
            return off + plsc.all_reduce_population_count(ok)[0]
        cnt = lax.fori_loop(0, EPH // 16, _cmp, jnp.int32(0), unroll=4)

        nblk = (cnt + K - 1) // K
        padn = nblk * K - cnt
        for t in range(K // 16):
            @pl.when(t * 16 < padn)
            def _(t=t):
                src_v[pl.ds(cnt + t * 16, 16)] = N + lane + 16 * t
                dst_v[pl.ds(cnt + t * 16, 16)] = NH + lane + 16 * t

        def _rl(r, _):
            def _rc(t, _):
                dst2_v[r, pl.ds(t * 16, 16)] = \
                    dst_v[pl.ds(r * K + t * 16, 16)]
                return 0
            lax.fori_loop(0, K // 16, _rc, 0)
            return 0
        lax.fori_loop(0, nblk, _rl, 0)

        @pl.when(nblk > 0)
        def _():
            pltpu.async_copy(m_hbm.at[src_v.at[pl.ds(0, K)]],
                             rows2.at[0], sems.at[0])

        def _blk(j, _):
            p = lax.rem(j, 3)
            pn = lax.rem(j + 1, 3)

            @pl.when(j + 1 < nblk)
            def _():
                pltpu.async_copy(
                    m_hbm.at[src_v.at[pl.ds((j + 1) * K, K)]],
                    rows2.at[pn], sems.at[pn])

            def _hist(t, _):
                idx = dst2_v[j, pl.ds(t * 16, 16)]
                plsc.addupdate_scatter(hist_v, [idx], one16,
                                       mask=idx < NH)
                return 0
            lax.fori_loop(0, K // 16, _hist, 0)

            pltpu.make_async_copy(m_hbm.at[src_v.at[pl.ds(j * K, K)]],
                                  rows2.at[p], sems.at[p]).wait()
            pltpu.sync_copy(rows2.at[p], zsh.at[dst2_v.at[j]], add=True)
            return 0
        lax.fori_loop(0, nblk, _blk, 0)
        return 0
    lax.fori_loop(0, 2, _half, 0)

    plsc.subcore_barrier()

    sl = pl.ds(s * RPC, RPC)
    pltpu.sync_copy(zsh.at[sl], z_out.at[pl.ds(base + s * RPC, RPC)])
    pltpu.sync_copy(hist_v, deg_out.at[c, s])


def _sc_edge_pass(m_pad, ei):
    mesh = plsc.VectorSubcoreMesh(
        core_axis_name="c", subcore_axis_name="s",
        num_cores=NC, num_subcores=NS,
    )
    return pl.kernel(
        _sc_edge_body,
        out_type=(
            jax.ShapeDtypeStruct((N_PAD, D), jnp.float32),
            jax.ShapeDtypeStruct((NC, NS, NH), jnp.float32),
        ),
        mesh=mesh,
        scratch_types=[
            pltpu.VMEM((EPH + 16,), jnp.int32),
            pltpu.VMEM((EPH + 16,), jnp.int32),
            pltpu.VMEM((NBH, K), jnp.int32),
            pltpu.VMEM((3, K, D), jnp.float32),
            pltpu.VMEM((NH,), jnp.float32),
            pltpu.VMEM_SHARED((ZROWS, D), jnp.float32),
            pltpu.SemaphoreType.DMA((3,)),
        ],
        compiler_params=pltpu.CompilerParams(needs_layout_passes=False),
    )(m_pad, ei)


def _update_body(z_ref, d_ref, x_ref, w_ref, b_ref, o_ref):
    ones_cols = jnp.ones((NS, D), jnp.float32)
    deg_mat = lax.dot_general(
        d_ref[0], ones_cols, (((0,), (0,)), ((), ())),
        preferred_element_type=jnp.float32,
    )
    z = z_ref[:] / jnp.maximum(deg_mat, 1.0)
    o_ref[:] = jax.nn.relu(
        jnp.dot(z, w_ref[:], preferred_element_type=jnp.float32) + b_ref[:]
    ) + x_ref[:]


def _node_update(z, deg, x, Wu, bu2):
    return pl.pallas_call(
        _update_body,
        grid=(N_PAD // TCB,),
        in_specs=[
            pl.BlockSpec((TCB, D), lambda i: (i, 0)),
            pl.BlockSpec(
                (1, NS, TCB),
                lambda i: (i // (NH // TCB), 0, i % (NH // TCB)),
            ),
            pl.BlockSpec((TCB, D), lambda i: (i, 0)),
            pl.BlockSpec((D, D), lambda i: (0, 0)),
            pl.BlockSpec((1, D), lambda i: (0, 0)),
        ],
        out_specs=pl.BlockSpec((TCB, D), lambda i: (i, 0)),
        out_shape=jax.ShapeDtypeStruct((N, D), jnp.float32),
    )(z, deg, x, Wu, bu2)


def kernel(x, edge_index, Wp, bp, Wu, bu):
    x = x.astype(jnp.float32)
    ei = edge_index.astype(jnp.int32)

    m_pad = _msg_precompute(x, Wp, bp.reshape(1, D))
    z, deg = _sc_edge_pass(m_pad, ei)
    return _node_update(z, deg, x, Wu, bu.reshape(1, D))
